# Initial kernel scaffold; baseline (speedup 1.0000x reference)
import jax
import jax.numpy as jnp
from jax.experimental import pallas as pl


def kernel(inputs):
    raise NotImplementedError("write your pallas kernel here")


# speedup vs baseline: 2.2361x; 2.2361x over previous
import functools

import jax
import jax.numpy as jnp
from jax import lax
from jax.experimental import pallas as pl
from jax.experimental.pallas import tpu as pltpu
from jax.experimental.pallas import tpu_sc as plsc

R = 128
N = 32768
K = 64
L = 16
NW = 32
ROWS_PER_W = R // NW
NCHUNK = N // L
NBINS = 256
HIST_WORDS = NBINS * L


def _sc_row_kth(x):
    mesh = plsc.VectorSubcoreMesh(core_axis_name="c", subcore_axis_name="s")

    @functools.partial(
        pl.kernel,
        mesh=mesh,
        out_type=jax.ShapeDtypeStruct((NW, L), jnp.int32),
        compiler_params=pltpu.CompilerParams(needs_layout_passes=False),
        scratch_types=[
            pltpu.VMEM((N,), jnp.int32),
            pltpu.VMEM((N,), jnp.int32),
            pltpu.VMEM((N,), jnp.int32),
            pltpu.VMEM((HIST_WORDS,), jnp.int32),
            pltpu.VMEM((L,), jnp.int32),
            pltpu.SemaphoreType.DMA,
            pltpu.SemaphoreType.DMA,
        ],
    )
    def k(x_hbm, out_hbm, buf0, buf1, keybuf, hist, stage, sem0, sem1):
        w = lax.axis_index("s") * 2 + lax.axis_index("c")
        row0 = w * ROWS_PER_W
        iota = lax.iota(jnp.int32, L)
        ones = jnp.ones((L,), jnp.int32)
        zeros = jnp.zeros((L,), jnp.int32)

        def zero_hist():
            def zb(i, _):
                hist[pl.ds(i * L, L)] = zeros
                return 0
            lax.fori_loop(0, NBINS, zb, 0)

        def merge_group(j):
            h = zeros
            base = j * (L * L)
            for t in range(L):
                h = h + plsc.load_gather(hist, [base + iota * L + t])
            return h

        def scan_hist(kk):
            def bodyA(jj, st):
                best, carry = st
                j = 15 - jj
                h = merge_group(j)
                cs = plsc.cumsum(h)
                tot = jnp.max(cs)
                cnt = (carry + tot) - cs + h
                bvec = j * L + iota
                cand = jnp.where(cnt >= kk, bvec, -1)
                return jnp.maximum(best, jnp.max(cand)), carry + tot

            best, _ = lax.fori_loop(
                0, 16, bodyA, (jnp.int32(-1), jnp.int32(0)))

            def bodyB(j, accv):
                h = merge_group(j)
                bvec = j * L + iota
                return accv + jnp.where(bvec > best, h, 0)

            accv = lax.fori_loop(0, 16, bodyB, zeros)
            return best, jnp.sum(accv)

        def process_row(buf):
            zero_hist()

            def p0(i, _):
                u = buf[pl.ds(i * L, L)]
                sgn = jnp.right_shift(u, 31)
                key = u ^ (sgn & 0x7FFFFFFF)
                keybuf[pl.ds(i * L, L)] = key
                b = jnp.right_shift(key, 24) + 128
                plsc.addupdate_scatter(hist, [b * L + iota], ones)
                return 0

            lax.fori_loop(0, NCHUNK, p0, 0)
            b0, s0 = scan_hist(jnp.int32(K))
            prefix = jnp.left_shift(b0 - 128, 24)
            kk = jnp.int32(K) - s0

            for shift in (16, 8, 0):
                zero_hist()
                himask = jnp.int32(-(1 << (shift + 8)))

                def pm(i, _, himask=himask, shift=shift, prefix=prefix):
                    key = keybuf[pl.ds(i * L, L)]
                    match = (key & himask) == prefix
                    b = jnp.right_shift(key, shift) & 0xFF
                    plsc.addupdate_scatter(
                        hist, [b * L + iota], ones, mask=match)
                    return 0

                lax.fori_loop(0, NCHUNK, pm, 0)
                bl, sl = scan_hist(kk)
                prefix = prefix | jnp.left_shift(bl, shift)
                kk = kk - sl

            return jnp.broadcast_to(prefix, (L,))

        bufs = (buf0, buf1)
        sems = (sem0, sem1)
        cps = [None] * ROWS_PER_W
        cps[0] = pltpu.make_async_copy(x_hbm.at[row0], buf0, sem0)
        cps[0].start()
        acc = jnp.full((L,), 0x7FFFFFFF, jnp.int32)
        for r in range(ROWS_PER_W):
            if r + 1 < ROWS_PER_W:
                cps[r + 1] = pltpu.make_async_copy(
                    x_hbm.at[row0 + (r + 1)], bufs[(r + 1) % 2],
                    sems[(r + 1) % 2])
                cps[r + 1].start()
            cps[r].wait()
            acc = jnp.minimum(acc, process_row(bufs[r % 2]))

        stage[...] = acc
        pltpu.sync_copy(stage, out_hbm.at[w])

    return k(x)


def _tc_mask(x, kv8):
    def body(kv_ref, x_ref, o_ref):
        kmin = jnp.min(kv_ref[...])
        sgn = jnp.right_shift(kmin, 31)
        t = lax.bitcast_convert_type(kmin ^ (sgn & 0x7FFFFFFF), jnp.float32)
        xv = x_ref[...]
        o_ref[...] = jnp.where(xv >= t, xv, 0.0)

    grid = (16,)
    return pl.pallas_call(
        body,
        grid=grid,
        in_specs=[
            pl.BlockSpec((4, 128), lambda i: (0, 0)),
            pl.BlockSpec((8, N), lambda i: (i, 0)),
        ],
        out_specs=pl.BlockSpec((8, N), lambda i: (i, 0)),
        out_shape=jax.ShapeDtypeStruct((R, N), jnp.float32),
    )(kv8, x)


def kernel(inputs):
    xi = lax.bitcast_convert_type(inputs, jnp.int32)
    kv = _sc_row_kth(xi)
    kv8 = kv.reshape(4, 128)
    return _tc_mask(inputs, kv8)


# speedup vs baseline: 7.6746x; 3.4321x over previous
import functools

import jax
import jax.numpy as jnp
from jax import lax
from jax.experimental import pallas as pl
from jax.experimental.pallas import tpu as pltpu
from jax.experimental.pallas import tpu_sc as plsc

R = 128
N = 32768
K = 64
L = 16
NW = 32
ROWS_PER_W = R // NW
NCHUNK = N // L
NBINS = 256
HIST_WORDS = NBINS * L


def _sc_row_kth(x):
    mesh = plsc.VectorSubcoreMesh(core_axis_name="c", subcore_axis_name="s")

    @functools.partial(
        pl.kernel,
        mesh=mesh,
        out_type=jax.ShapeDtypeStruct((NW, L), jnp.int32),
        compiler_params=pltpu.CompilerParams(needs_layout_passes=False),
        scratch_types=[
            pltpu.VMEM((N,), jnp.int32),
            pltpu.VMEM((N,), jnp.int32),
            pltpu.VMEM((N,), jnp.int32),
            pltpu.VMEM((HIST_WORDS,), jnp.int32),
            pltpu.VMEM((L,), jnp.int32),
            pltpu.SemaphoreType.DMA,
            pltpu.SemaphoreType.DMA,
        ],
    )
    def k(x_hbm, out_hbm, buf0, buf1, keybuf, hist, stage, sem0, sem1):
        w = lax.axis_index("s") * 2 + lax.axis_index("c")
        row0 = w * ROWS_PER_W
        iota = lax.iota(jnp.int32, L)
        iota2048 = iota + NBINS * 8
        ones = jnp.ones((L,), jnp.int32)
        zeros = jnp.zeros((L,), jnp.int32)

        def zero_hist():
            @plsc.parallel_loop(0, NBINS, unroll=16)
            def _(i):
                hist[pl.ds(i * L, L)] = zeros

        def merge_group(j):
            h = zeros
            base = j * (L * L)
            for t in range(L):
                h = h + plsc.load_gather(hist, [base + iota * L + t])
            return h

        def scan_hist(kk):
            def bodyA(jj, st):
                best, carry = st
                j = 15 - jj
                h = merge_group(j)
                cs = plsc.cumsum(h)
                tot = jnp.max(cs)
                cnt = (carry + tot) - cs + h
                bvec = j * L + iota
                cand = jnp.where(cnt >= kk, bvec, -1)
                return jnp.maximum(best, jnp.max(cand)), carry + tot

            best, _ = lax.fori_loop(
                0, 16, bodyA, (jnp.int32(-1), jnp.int32(0)))

            def bodyB(j, accv):
                h = merge_group(j)
                bvec = j * L + iota
                return accv + jnp.where(bvec > best, h, 0)

            accv = lax.fori_loop(0, 16, bodyB, zeros)
            return best, jnp.sum(accv)

        def process_row(buf):
            zero_hist()

            @plsc.parallel_loop(0, NCHUNK, unroll=8)
            def _(i):
                u = buf[pl.ds(i * L, L)]
                sgn = jnp.right_shift(u, 31)
                key = u ^ (sgn & 0x7FFFFFFF)
                keybuf[pl.ds(i * L, L)] = key
                idx = (jnp.right_shift(key, 20) & -16) + iota2048
                plsc.addupdate_scatter(hist, [idx], ones)
            b0, s0 = scan_hist(jnp.int32(K))
            prefix = jnp.left_shift(b0 - 128, 24)
            kk = jnp.int32(K) - s0

            for shift in (16, 8, 0):
                zero_hist()
                himask = jnp.int32(-(1 << (shift + 8)))

                @plsc.parallel_loop(0, NCHUNK, unroll=8)
                def _(i, himask=himask, shift=shift, prefix=prefix):
                    key = keybuf[pl.ds(i * L, L)]
                    match = (key & himask) == prefix
                    b = jnp.right_shift(key, shift) & 0xFF
                    plsc.addupdate_scatter(
                        hist, [b * L + iota], ones, mask=match)
                bl, sl = scan_hist(kk)
                prefix = prefix | jnp.left_shift(bl, shift)
                kk = kk - sl

            return jnp.broadcast_to(prefix, (L,))

        bufs = (buf0, buf1)
        sems = (sem0, sem1)
        cps = [None] * ROWS_PER_W
        cps[0] = pltpu.make_async_copy(x_hbm.at[row0], buf0, sem0)
        cps[0].start()
        acc = jnp.full((L,), 0x7FFFFFFF, jnp.int32)
        for r in range(ROWS_PER_W):
            if r + 1 < ROWS_PER_W:
                cps[r + 1] = pltpu.make_async_copy(
                    x_hbm.at[row0 + (r + 1)], bufs[(r + 1) % 2],
                    sems[(r + 1) % 2])
                cps[r + 1].start()
            cps[r].wait()
            acc = jnp.minimum(acc, process_row(bufs[r % 2]))

        stage[...] = acc
        pltpu.sync_copy(stage, out_hbm.at[w])

    return k(x)


def _tc_mask(x, kv8):
    def body(kv_ref, x_ref, o_ref):
        kmin = jnp.min(kv_ref[...])
        sgn = jnp.right_shift(kmin, 31)
        t = lax.bitcast_convert_type(kmin ^ (sgn & 0x7FFFFFFF), jnp.float32)
        xv = x_ref[...]
        o_ref[...] = jnp.where(xv >= t, xv, 0.0)

    grid = (16,)
    return pl.pallas_call(
        body,
        grid=grid,
        in_specs=[
            pl.BlockSpec((4, 128), lambda i: (0, 0)),
            pl.BlockSpec((8, N), lambda i: (i, 0)),
        ],
        out_specs=pl.BlockSpec((8, N), lambda i: (i, 0)),
        out_shape=jax.ShapeDtypeStruct((R, N), jnp.float32),
    )(kv8, x)


def kernel(inputs):
    xi = lax.bitcast_convert_type(inputs, jnp.int32)
    kv = _sc_row_kth(xi)
    kv8 = kv.reshape(4, 128)
    return _tc_mask(inputs, kv8)


# speedup vs baseline: 8.7409x; 1.1389x over previous
import functools

import jax
import jax.numpy as jnp
from jax import lax
from jax.experimental import pallas as pl
from jax.experimental.pallas import tpu as pltpu
from jax.experimental.pallas import tpu_sc as plsc

R = 128
N = 32768
K = 64
L = 16
NW = 32
ROWS_PER_W = R // NW
NCHUNK = N // L
NBINS = 256
HIST_WORDS = NBINS * L
CAND_KEY_MIN = 0x40000000
THRESH_BIN = 192


def _sc_row_kth(x):
    mesh = plsc.VectorSubcoreMesh(core_axis_name="c", subcore_axis_name="s")

    @functools.partial(
        pl.kernel,
        mesh=mesh,
        out_type=jax.ShapeDtypeStruct((NW, L), jnp.int32),
        compiler_params=pltpu.CompilerParams(needs_layout_passes=False),
        scratch_types=[
            pltpu.VMEM((N,), jnp.int32),
            pltpu.VMEM((N,), jnp.int32),
            pltpu.VMEM((N,), jnp.int32),
            pltpu.VMEM((HIST_WORDS,), jnp.int32),
            pltpu.VMEM((L,), jnp.int32),
            pltpu.SemaphoreType.DMA,
            pltpu.SemaphoreType.DMA,
        ],
    )
    def k(x_hbm, out_hbm, buf0, buf1, cand, hist, stage, sem0, sem1):
        w = lax.axis_index("s") * 2 + lax.axis_index("c")
        row0 = w * ROWS_PER_W
        iota = lax.iota(jnp.int32, L)
        iota2048 = iota + NBINS * 8
        ones = jnp.ones((L,), jnp.int32)
        zeros = jnp.zeros((L,), jnp.int32)

        def zero_hist():
            @plsc.parallel_loop(0, NBINS, unroll=16)
            def _(i):
                hist[pl.ds(i * L, L)] = zeros

        def merge_group(j):
            h = zeros
            base = j * (L * L)
            for t in range(L):
                h = h + plsc.load_gather(hist, [base + iota * L + t])
            return h

        def scan_hist(kk):
            def bodyA(jj, st):
                best, carry = st
                j = 15 - jj
                h = merge_group(j)
                cs = plsc.cumsum(h)
                tot = jnp.max(cs)
                cnt = (carry + tot) - cs + h
                bvec = j * L + iota
                cand = jnp.where(cnt >= kk, bvec, -1)
                return jnp.maximum(best, jnp.max(cand)), carry + tot

            best, _ = lax.fori_loop(
                0, 16, bodyA, (jnp.int32(-1), jnp.int32(0)))

            def bodyB(j, accv):
                h = merge_group(j)
                bvec = j * L + iota
                return accv + jnp.where(bvec > best, h, 0)

            accv = lax.fori_loop(0, 16, bodyB, zeros)
            return best, jnp.sum(accv)

        def process_row(buf):
            zero_hist()

            @plsc.parallel_loop(0, NCHUNK, unroll=8, carry=zeros)
            def offv(i, off):
                u = buf[pl.ds(i * L, L)]
                sgn = jnp.right_shift(u, 31)
                key = u ^ (sgn & 0x7FFFFFFF)
                idx = (jnp.right_shift(key, 20) & -16) + iota2048
                plsc.addupdate_scatter(hist, [idx], ones)
                cm = key >= CAND_KEY_MIN
                plsc.store_scatter(cand, [off * L + iota], key, mask=cm)
                return off + jnp.where(cm, 1, 0)

            b0, s0 = scan_hist(jnp.int32(K))
            prefix = jnp.left_shift(b0 - 128, 24)
            kk = jnp.int32(K) - s0

            def fb_collect(prefix=prefix):
                @plsc.parallel_loop(0, NCHUNK, unroll=8, carry=zeros)
                def off2(i, off):
                    u = buf[pl.ds(i * L, L)]
                    sgn = jnp.right_shift(u, 31)
                    key = u ^ (sgn & 0x7FFFFFFF)
                    m = (key & jnp.int32(-(1 << 24))) == prefix
                    plsc.store_scatter(cand, [off * L + iota], key, mask=m)
                    return off + jnp.where(m, 1, 0)
                return off2

            offv = lax.cond(b0 >= THRESH_BIN, lambda: offv, fb_collect)

            for shift in (16, 8, 0):
                zero_hist()
                himask = jnp.int32(-(1 << (shift + 8)))
                pmax = jnp.max(offv)

                def sv(p, _, himask=himask, shift=shift, prefix=prefix,
                       offv=offv):
                    kv = cand[pl.ds(p * L, L)]
                    m = (p < offv) & ((kv & himask) == prefix)
                    b = jnp.right_shift(kv, shift) & 0xFF
                    plsc.addupdate_scatter(
                        hist, [b * L + iota], ones, mask=m)
                    return 0

                lax.fori_loop(0, pmax, sv, 0)
                bl, sl = scan_hist(kk)
                prefix = prefix | jnp.left_shift(bl, shift)
                kk = kk - sl

            return jnp.broadcast_to(prefix, (L,))

        bufs = (buf0, buf1)
        sems = (sem0, sem1)
        cps = [None] * ROWS_PER_W
        cps[0] = pltpu.make_async_copy(x_hbm.at[row0], buf0, sem0)
        cps[0].start()
        acc = jnp.full((L,), 0x7FFFFFFF, jnp.int32)
        for r in range(ROWS_PER_W):
            if r + 1 < ROWS_PER_W:
                cps[r + 1] = pltpu.make_async_copy(
                    x_hbm.at[row0 + (r + 1)], bufs[(r + 1) % 2],
                    sems[(r + 1) % 2])
                cps[r + 1].start()
            cps[r].wait()
            acc = jnp.minimum(acc, process_row(bufs[r % 2]))

        stage[...] = acc
        pltpu.sync_copy(stage, out_hbm.at[w])

    return k(x)


def _tc_mask(x, kv8):
    def body(kv_ref, x_ref, o_ref):
        kmin = jnp.min(kv_ref[...])
        sgn = jnp.right_shift(kmin, 31)
        t = lax.bitcast_convert_type(kmin ^ (sgn & 0x7FFFFFFF), jnp.float32)
        xv = x_ref[...]
        o_ref[...] = jnp.where(xv >= t, xv, 0.0)

    grid = (16,)
    return pl.pallas_call(
        body,
        grid=grid,
        in_specs=[
            pl.BlockSpec((4, 128), lambda i: (0, 0)),
            pl.BlockSpec((8, N), lambda i: (i, 0)),
        ],
        out_specs=pl.BlockSpec((8, N), lambda i: (i, 0)),
        out_shape=jax.ShapeDtypeStruct((R, N), jnp.float32),
    )(kv8, x)


def kernel(inputs):
    xi = lax.bitcast_convert_type(inputs, jnp.int32)
    kv = _sc_row_kth(xi)
    kv8 = kv.reshape(4, 128)
    return _tc_mask(inputs, kv8)


# speedup vs baseline: 11.4534x; 1.3103x over previous
import functools

import jax
import jax.numpy as jnp
from jax import lax
from jax.experimental import pallas as pl
from jax.experimental.pallas import tpu as pltpu
from jax.experimental.pallas import tpu_sc as plsc

R = 128
N = 32768
K = 64
L = 16
NW = 32
ROWS_PER_W = R // NW
NCHUNK = N // L
NBINS = 256
HIST_WORDS = NBINS * L
MININT = -(1 << 31)
CAND_KEY_MIN = 0x40000000


def _sc_row_kth(x):
    mesh = plsc.VectorSubcoreMesh(core_axis_name="c", subcore_axis_name="s")

    @functools.partial(
        pl.kernel,
        mesh=mesh,
        out_type=jax.ShapeDtypeStruct((NW, L), jnp.int32),
        compiler_params=pltpu.CompilerParams(needs_layout_passes=False),
        scratch_types=[
            pltpu.VMEM((N,), jnp.float32),
            pltpu.VMEM((N,), jnp.float32),
            pltpu.VMEM((N,), jnp.int32),
            pltpu.VMEM((HIST_WORDS,), jnp.int32),
            pltpu.VMEM((L,), jnp.int32),
            pltpu.SemaphoreType.DMA,
            pltpu.SemaphoreType.DMA,
        ],
    )
    def k(x_hbm, out_hbm, buf0, buf1, cand, hist, stage, sem0, sem1):
        w = lax.axis_index("s") * 2 + lax.axis_index("c")
        row0 = w * ROWS_PER_W
        iota = lax.iota(jnp.int32, L)
        ones = jnp.ones((L,), jnp.int32)
        zeros = jnp.zeros((L,), jnp.int32)

        def zero_hist():
            @plsc.parallel_loop(0, NBINS, unroll=16)
            def _(i):
                hist[pl.ds(i * L, L)] = zeros

        def merge_group(j):
            h = zeros
            base = j * (L * L)
            for t in range(L):
                h = h + plsc.load_gather(hist, [base + iota * L + t])
            return h

        def scan_hist(kk):
            def bodyA(jj, st):
                enc_best, carry = st
                j = 15 - jj
                h = merge_group(j)
                cs = plsc.cumsum(h)
                tot = jnp.max(cs)
                cnt = (carry + tot) - cs + h
                above = jnp.minimum(cnt - h, 255)
                bvec = j * L + iota
                enc = jnp.where(
                    cnt >= kk, jnp.left_shift(bvec, 8) + (255 - above), -1)
                return jnp.maximum(enc_best, jnp.max(enc)), carry + tot

            enc, _ = lax.fori_loop(
                0, 16, bodyA, (jnp.int32(-1), jnp.int32(0)))
            return jnp.right_shift(enc, 8), 255 - (enc & 255)

        def transform(v):
            u = plsc.bitcast(v, jnp.int32)
            sgn = jnp.right_shift(u, 31)
            key_s = u ^ (sgn & 0x7FFFFFFF)
            return key_s, key_s ^ MININT

        def process_row(buf):
            @plsc.parallel_loop(0, NCHUNK, unroll=8, carry=zeros)
            def offv(i, off):
                key_s, kb = transform(buf[pl.ds(i * L, L)])
                cm = key_s >= CAND_KEY_MIN
                plsc.store_scatter(cand, [off * L + iota], kb, mask=cm)
                return off + jnp.where(cm, 1, 0)

            m = jnp.sum(offv)

            def fb():
                @plsc.parallel_loop(0, NCHUNK, unroll=8)
                def _(i):
                    _, kb = transform(buf[pl.ds(i * L, L)])
                    cand[pl.ds(i * L, L)] = kb
                return jnp.full((L,), NCHUNK, jnp.int32)

            offv = lax.cond(m >= K, lambda: offv, fb)
            pmax = jnp.max(offv)

            kk = jnp.int32(K)
            prefix = jnp.int32(0)
            for shift in (24, 16, 8, 0):
                zero_hist()
                himask = jnp.int32(-(1 << (shift + 8))) if shift < 24 else 0

                @plsc.parallel_loop(0, pmax, unroll=2)
                def _(p, himask=himask, shift=shift, prefix=prefix,
                      offv=offv):
                    kv = cand[pl.ds(p * L, L)]
                    valid = (p < offv) & ((kv & himask) == prefix)
                    b = lax.shift_right_logical(kv, shift) & 0xFF
                    plsc.addupdate_scatter(
                        hist, [b * L + iota], ones, mask=valid)

                bl, sl = scan_hist(kk)
                prefix = prefix | jnp.left_shift(bl, shift)
                kk = kk - sl

            return jnp.broadcast_to(prefix ^ MININT, (L,))

        bufs = (buf0, buf1)
        sems = (sem0, sem1)
        cps = [None] * ROWS_PER_W
        cps[0] = pltpu.make_async_copy(x_hbm.at[row0], buf0, sem0)
        cps[0].start()
        acc = jnp.full((L,), 0x7FFFFFFF, jnp.int32)
        for r in range(ROWS_PER_W):
            if r + 1 < ROWS_PER_W:
                cps[r + 1] = pltpu.make_async_copy(
                    x_hbm.at[row0 + (r + 1)], bufs[(r + 1) % 2],
                    sems[(r + 1) % 2])
                cps[r + 1].start()
            cps[r].wait()
            acc = jnp.minimum(acc, process_row(bufs[r % 2]))

        stage[...] = acc
        pltpu.sync_copy(stage, out_hbm.at[w])

    return k(x)


def _tc_mask(x, kv8):
    def body(kv_ref, x_ref, o_ref):
        kmin = jnp.min(kv_ref[...])
        sgn = jnp.right_shift(kmin, 31)
        t = lax.bitcast_convert_type(kmin ^ (sgn & 0x7FFFFFFF), jnp.float32)
        xv = x_ref[...]
        o_ref[...] = jnp.where(xv >= t, xv, 0.0)

    grid = (8,)
    return pl.pallas_call(
        body,
        grid=grid,
        in_specs=[
            pl.BlockSpec((4, 128), lambda i: (0, 0)),
            pl.BlockSpec((16, N), lambda i: (i, 0)),
        ],
        out_specs=pl.BlockSpec((16, N), lambda i: (i, 0)),
        out_shape=jax.ShapeDtypeStruct((R, N), jnp.float32),
    )(kv8, x)


def kernel(inputs):
    kv = _sc_row_kth(inputs)
    kv8 = kv.reshape(4, 128)
    return _tc_mask(inputs, kv8)
